# baseline (speedup 1.0000x reference)
import functools

import jax
import jax.numpy as jnp
from jax import lax
from jax.experimental import pallas as pl
from jax.experimental.pallas import tpu as pltpu
from jax.experimental.pallas import tpu_sc as plsc

N_NODES = 10000
N_EDGES = 320000
D_FEAT = 128
HW = D_FEAT // 2

NC = 2
NS = 16

EPW = N_EDGES // NS
K = 125
NCHUNK = 160
G = 8
NG = NCHUNK // G
NIB = 4
CPI = NIB * G
NITER = NCHUNK // CPI
N_ACC = N_NODES + 16
NBUF = 4
WRB = 640
WRB_LAST = N_NODES - (NS - 1) * WRB
DEG_W = 8

_mesh = plsc.VectorSubcoreMesh(core_axis_name="c", subcore_axis_name="s")


@functools.partial(
    pl.kernel,
    out_type=(
        jax.ShapeDtypeStruct((NC, N_NODES, HW), jnp.float32),
        jax.ShapeDtypeStruct((NC, N_NODES, DEG_W), jnp.float32),
    ),
    mesh=_mesh,
    compiler_params=pltpu.CompilerParams(use_tc_tiling_on_sc=False),
    scratch_types=[
        [pltpu.VMEM((G, K), jnp.int32) for _ in range(NIB)],
        [pltpu.VMEM((G, K), jnp.int32) for _ in range(NIB)],
        [pltpu.VMEM((K, HW), jnp.float32) for _ in range(NBUF)],
        pltpu.VMEM((K, DEG_W), jnp.float32),
        pltpu.VMEM_SHARED((N_NODES, HW), jnp.float32),
        pltpu.VMEM_SHARED((N_ACC, HW), jnp.float32),
        pltpu.VMEM_SHARED((N_ACC, DEG_W), jnp.float32),
        [pltpu.SemaphoreType.DMA for _ in range(NIB)],
        [pltpu.SemaphoreType.DMA for _ in range(NIB)],
        [pltpu.SemaphoreType.DMA for _ in range(NBUF)],
        [pltpu.SemaphoreType.DMA for _ in range(NBUF)],
        [pltpu.SemaphoreType.DMA for _ in range(NBUF)],
    ],
)
def _sc_agg(x_hbm, e_hbm, zrow_hbm, zdeg_hbm, ones_hbm,
            out_hbm, deg_out_hbm,
            ib_s, ib_d, rows_v, ones_v, x_sh, acc_sh, deg_sh,
            sem_is, sem_id, sem_g, sem_s, sem_d):
    c = lax.axis_index("c")
    s = lax.axis_index("s")

    pltpu.sync_copy(ones_hbm, ones_v)

    @pl.when(s < NS - 1)
    def _():
        pltpu.sync_copy(x_hbm.at[pl.ds(s * WRB, WRB), c],
                        x_sh.at[pl.ds(s * WRB, WRB)])
        pltpu.sync_copy(zrow_hbm, acc_sh.at[pl.ds(s * WRB, WRB)])
        pltpu.sync_copy(zdeg_hbm, deg_sh.at[pl.ds(s * WRB, WRB)])

    @pl.when(s == NS - 1)
    def _():
        pltpu.sync_copy(x_hbm.at[pl.ds((NS - 1) * WRB, WRB_LAST), c],
                        x_sh.at[pl.ds((NS - 1) * WRB, WRB_LAST)])
        pltpu.sync_copy(zrow_hbm.at[pl.ds(0, WRB_LAST)],
                        acc_sh.at[pl.ds((NS - 1) * WRB, WRB_LAST)])
        pltpu.sync_copy(zdeg_hbm.at[pl.ds(0, WRB_LAST)],
                        deg_sh.at[pl.ds((NS - 1) * WRB, WRB_LAST)])

    plsc.subcore_barrier()

    def idx_copies(g, b):
        return (
            pltpu.make_async_copy(e_hbm.at[s * NG + g], ib_s[b], sem_is[b]),
            pltpu.make_async_copy(e_hbm.at[(NS + s) * NG + g], ib_d[b],
                                  sem_id[b]),
        )

    def idx_start(g, b):
        for cp in idx_copies(g, b):
            cp.start()

    def idx_wait(g, b):
        for cp in idx_copies(g, b):
            cp.wait()

    def gather(i, t):
        b, u = i // G, i % NBUF
        pltpu.async_copy(x_sh.at[ib_s[b].at[i % G]], rows_v[u], sem_g[u])

    def gwait(i, t):
        b, u = i // G, i % NBUF
        pltpu.make_async_copy(x_sh.at[ib_s[b].at[i % G]], rows_v[u],
                              sem_g[u]).wait()

    def deg_on(i):
        return (c == 0) if i % 2 == 0 else (c != 0)

    def scat(i, t):
        b, u = i // G, i % NBUF
        pltpu.async_copy(rows_v[u], acc_sh.at[ib_d[b].at[i % G]],
                         sem_s[u], add=True)

        @pl.when(deg_on(i))
        def _():
            pltpu.async_copy(ones_v, deg_sh.at[ib_d[b].at[i % G]],
                             sem_d[u], add=True)

    def swait(i, t):
        b, u = i // G, i % NBUF
        pltpu.make_async_copy(rows_v[u], acc_sh.at[ib_d[b].at[i % G]],
                              sem_s[u]).wait()

        @pl.when(deg_on(i))
        def _():
            pltpu.make_async_copy(ones_v, deg_sh.at[ib_d[b].at[i % G]],
                                  sem_d[u]).wait()

    for b0 in range(NIB - 1):
        idx_start(b0, b0)

    def body(t, carry):
        for b in range(NIB - 1):
            idx_wait(t * NIB + b, b)

        for i0 in range(NBUF - 1):
            gather(i0, t)

        for i in range(CPI):
            gwait(i, t)
            scat(i, t)

            if i == 0:
                @pl.when(t >= 1)
                def _():
                    swait(CPI - 1, t)

                idx_start(t * NIB + (NIB - 1), NIB - 1)
            else:
                swait(i - 1, t)

            if i == 20:
                idx_wait(t * NIB + (NIB - 1), NIB - 1)

            if i + NBUF - 1 < CPI:
                gather(i + NBUF - 1, t)

        @pl.when(t < NITER - 1)
        def _():
            for b in range(NIB - 1):
                idx_start((t + 1) * NIB + b, b)

        return carry

    lax.fori_loop(0, NITER, body, 0)

    swait(CPI - 1, NITER - 1)

    plsc.subcore_barrier()

    @pl.when(s < NS - 1)
    def _():
        pltpu.sync_copy(acc_sh.at[pl.ds(s * WRB, WRB)],
                        out_hbm.at[c, pl.ds(s * WRB, WRB)])
        pltpu.sync_copy(deg_sh.at[pl.ds(s * WRB, WRB)],
                        deg_out_hbm.at[c, pl.ds(s * WRB, WRB)])

    @pl.when(s == NS - 1)
    def _():
        pltpu.sync_copy(acc_sh.at[pl.ds((NS - 1) * WRB, WRB_LAST)],
                        out_hbm.at[c, pl.ds((NS - 1) * WRB, WRB_LAST)])
        pltpu.sync_copy(deg_sh.at[pl.ds((NS - 1) * WRB, WRB_LAST)],
                        deg_out_hbm.at[c, pl.ds((NS - 1) * WRB, WRB_LAST)])


_ROWS_BLK = 1000


def _combine_body(p_ref, d_ref, o_ref):
    deg = d_ref[0, :, 0] + d_ref[1, :, 0]
    inv = 1.0 / jnp.clip(deg, 1.0, None)[:, None]
    o_ref[...] = jnp.concatenate([p_ref[0], p_ref[1]], axis=-1) * inv


def _combine(partial, deg8):
    return pl.pallas_call(
        _combine_body,
        out_shape=jax.ShapeDtypeStruct((N_NODES, D_FEAT), jnp.float32),
        grid=(N_NODES // _ROWS_BLK,),
        in_specs=[
            pl.BlockSpec((NC, _ROWS_BLK, HW), lambda i: (0, i, 0)),
            pl.BlockSpec((NC, _ROWS_BLK, DEG_W), lambda i: (0, i, 0)),
        ],
        out_specs=pl.BlockSpec((_ROWS_BLK, D_FEAT), lambda i: (i, 0)),
    )(partial, deg8)


def kernel(x, edge_index):
    x2 = x.reshape(N_NODES, NC, HW)

    e_slab = edge_index.reshape(NC * NS * NG, G, K)

    zrow = jnp.zeros((WRB, HW), jnp.float32)
    zdeg = jnp.zeros((WRB, DEG_W), jnp.float32)
    ones = jnp.ones((K, DEG_W), jnp.float32)
    partial, deg8 = _sc_agg(x2, e_slab, zrow, zdeg, ones)
    return _combine(partial, deg8)


# speedup vs baseline: 1.0949x; 1.0949x over previous
import functools

import jax
import jax.numpy as jnp
from jax import lax
from jax.experimental import pallas as pl
from jax.experimental.pallas import tpu as pltpu
from jax.experimental.pallas import tpu_sc as plsc

N_NODES = 10000
N_EDGES = 320000
D_FEAT = 128
HW = D_FEAT // 2

NC = 2
NS = 16

EPW = N_EDGES // NS
K = 125
NCHUNK = 160
G = 8
NG = NCHUNK // G
NIB = 4
CPI = NIB * G
NITER = NCHUNK // CPI
N_ACC = N_NODES + 16
NBUF = 4
WRB = 640
WRB_LAST = N_NODES - (NS - 1) * WRB
DEG_W = 8

_mesh = plsc.VectorSubcoreMesh(core_axis_name="c", subcore_axis_name="s")


@functools.partial(
    pl.kernel,
    out_type=(
        jax.ShapeDtypeStruct((NC, N_NODES, HW), jnp.float32),
        jax.ShapeDtypeStruct((NC, N_NODES, DEG_W), jnp.float32),
    ),
    mesh=_mesh,
    compiler_params=pltpu.CompilerParams(use_tc_tiling_on_sc=False),
    scratch_types=[
        [pltpu.VMEM((G, K), jnp.int32) for _ in range(NIB)],
        [pltpu.VMEM((G, K), jnp.int32) for _ in range(NIB)],
        [pltpu.VMEM((K, HW), jnp.float32) for _ in range(NBUF)],
        pltpu.VMEM((K, DEG_W), jnp.float32),
        pltpu.VMEM_SHARED((N_NODES, HW), jnp.float32),
        pltpu.VMEM_SHARED((N_ACC, HW), jnp.float32),
        pltpu.VMEM_SHARED((N_ACC, DEG_W), jnp.float32),
        [pltpu.SemaphoreType.DMA for _ in range(NIB)],
        [pltpu.SemaphoreType.DMA for _ in range(NIB)],
        [pltpu.SemaphoreType.DMA for _ in range(NBUF)],
        [pltpu.SemaphoreType.DMA for _ in range(NBUF)],
        [pltpu.SemaphoreType.DMA for _ in range(NBUF)],
    ],
)
def _sc_agg(x_hbm, e_hbm, zrow_hbm, zdeg_hbm, ones_hbm,
            out_hbm, deg_out_hbm,
            ib_s, ib_d, rows_v, ones_v, x_sh, acc_sh, deg_sh,
            sem_is, sem_id, sem_g, sem_s, sem_d):
    c = lax.axis_index("c")
    s = lax.axis_index("s")

    pltpu.sync_copy(ones_hbm, ones_v)

    @pl.when(s < NS - 1)
    def _():
        pltpu.sync_copy(x_hbm.at[c, pl.ds(s * WRB, WRB)],
                        x_sh.at[pl.ds(s * WRB, WRB)])
        pltpu.sync_copy(zrow_hbm, acc_sh.at[pl.ds(s * WRB, WRB)])
        pltpu.sync_copy(zdeg_hbm, deg_sh.at[pl.ds(s * WRB, WRB)])

    @pl.when(s == NS - 1)
    def _():
        pltpu.sync_copy(x_hbm.at[c, pl.ds((NS - 1) * WRB, WRB_LAST)],
                        x_sh.at[pl.ds((NS - 1) * WRB, WRB_LAST)])
        pltpu.sync_copy(zrow_hbm.at[pl.ds(0, WRB_LAST)],
                        acc_sh.at[pl.ds((NS - 1) * WRB, WRB_LAST)])
        pltpu.sync_copy(zdeg_hbm.at[pl.ds(0, WRB_LAST)],
                        deg_sh.at[pl.ds((NS - 1) * WRB, WRB_LAST)])

    plsc.subcore_barrier()

    def idx_copies(g, b):
        return (
            pltpu.make_async_copy(e_hbm.at[s * NG + g], ib_s[b], sem_is[b]),
            pltpu.make_async_copy(e_hbm.at[(NS + s) * NG + g], ib_d[b],
                                  sem_id[b]),
        )

    def idx_start(g, b):
        for cp in idx_copies(g, b):
            cp.start()

    def idx_wait(g, b):
        for cp in idx_copies(g, b):
            cp.wait()

    def gather(i, t):
        b, u = i // G, i % NBUF
        pltpu.async_copy(x_sh.at[ib_s[b].at[i % G]], rows_v[u], sem_g[u])

    def gwait(i, t):
        b, u = i // G, i % NBUF
        pltpu.make_async_copy(x_sh.at[ib_s[b].at[i % G]], rows_v[u],
                              sem_g[u]).wait()

    def deg_on(i):
        return (c == 0) if i % 2 == 0 else (c != 0)

    def scat(i, t):
        b, u = i // G, i % NBUF
        pltpu.async_copy(rows_v[u], acc_sh.at[ib_d[b].at[i % G]],
                         sem_s[u], add=True)

        @pl.when(deg_on(i))
        def _():
            pltpu.async_copy(ones_v, deg_sh.at[ib_d[b].at[i % G]],
                             sem_d[u], add=True)

    def swait(i, t):
        b, u = i // G, i % NBUF
        pltpu.make_async_copy(rows_v[u], acc_sh.at[ib_d[b].at[i % G]],
                              sem_s[u]).wait()

        @pl.when(deg_on(i))
        def _():
            pltpu.make_async_copy(ones_v, deg_sh.at[ib_d[b].at[i % G]],
                                  sem_d[u]).wait()

    for b0 in range(NIB - 1):
        idx_start(b0, b0)

    def body(t, carry):
        for b in range(NIB - 1):
            idx_wait(t * NIB + b, b)

        for i0 in range(NBUF - 1):
            gather(i0, t)

        for i in range(CPI):
            gwait(i, t)
            scat(i, t)

            if i == 0:
                @pl.when(t >= 1)
                def _():
                    swait(CPI - 1, t)

                idx_start(t * NIB + (NIB - 1), NIB - 1)
            else:
                swait(i - 1, t)

            if i == 20:
                idx_wait(t * NIB + (NIB - 1), NIB - 1)

            if i + NBUF - 1 < CPI:
                gather(i + NBUF - 1, t)

        @pl.when(t < NITER - 1)
        def _():
            for b in range(NIB - 1):
                idx_start((t + 1) * NIB + b, b)

        return carry

    lax.fori_loop(0, NITER, body, 0)

    swait(CPI - 1, NITER - 1)

    plsc.subcore_barrier()

    @pl.when(s < NS - 1)
    def _():
        pltpu.sync_copy(acc_sh.at[pl.ds(s * WRB, WRB)],
                        out_hbm.at[c, pl.ds(s * WRB, WRB)])
        pltpu.sync_copy(deg_sh.at[pl.ds(s * WRB, WRB)],
                        deg_out_hbm.at[c, pl.ds(s * WRB, WRB)])

    @pl.when(s == NS - 1)
    def _():
        pltpu.sync_copy(acc_sh.at[pl.ds((NS - 1) * WRB, WRB_LAST)],
                        out_hbm.at[c, pl.ds((NS - 1) * WRB, WRB_LAST)])
        pltpu.sync_copy(deg_sh.at[pl.ds((NS - 1) * WRB, WRB_LAST)],
                        deg_out_hbm.at[c, pl.ds((NS - 1) * WRB, WRB_LAST)])


_ROWS_BLK = 2000


def _combine_body(p_ref, d_ref, o_ref):
    deg = d_ref[0, :, 0] + d_ref[1, :, 0]
    inv = 1.0 / jnp.clip(deg, 1.0, None)[:, None]
    o_ref[...] = jnp.concatenate([p_ref[0], p_ref[1]], axis=-1) * inv


def _combine(partial, deg8):
    return pl.pallas_call(
        _combine_body,
        out_shape=jax.ShapeDtypeStruct((N_NODES, D_FEAT), jnp.float32),
        grid=(N_NODES // _ROWS_BLK,),
        in_specs=[
            pl.BlockSpec((NC, _ROWS_BLK, HW), lambda i: (0, i, 0)),
            pl.BlockSpec((NC, _ROWS_BLK, DEG_W), lambda i: (0, i, 0)),
        ],
        out_specs=pl.BlockSpec((_ROWS_BLK, D_FEAT), lambda i: (i, 0)),
    )(partial, deg8)


def kernel(x, edge_index):
    x2 = x.reshape(N_NODES, NC, HW).transpose(1, 0, 2)

    e_slab = edge_index.reshape(NC * NS * NG, G, K)

    zrow = jnp.zeros((WRB, HW), jnp.float32)
    zdeg = jnp.zeros((WRB, DEG_W), jnp.float32)
    ones = jnp.ones((K, DEG_W), jnp.float32)
    partial, deg8 = _sc_agg(x2, e_slab, zrow, zdeg, ones)
    return _combine(partial, deg8)


# speedup vs baseline: 1.0964x; 1.0014x over previous
import functools

import jax
import jax.numpy as jnp
from jax import lax
from jax.experimental import pallas as pl
from jax.experimental.pallas import tpu as pltpu
from jax.experimental.pallas import tpu_sc as plsc

N_NODES = 10000
N_EDGES = 320000
D_FEAT = 128
HW = D_FEAT // 2

NC = 2
NS = 16

EPW = N_EDGES // NS
K = 125
NCHUNK = 160
G = 8
NG = NCHUNK // G
NIB = 4
CPI = NIB * G
NITER = NCHUNK // CPI
N_ACC = N_NODES + 16
NBUF = 4
WRB = 640
WRB_LAST = N_NODES - (NS - 1) * WRB
DEG_W = 8

_mesh = plsc.VectorSubcoreMesh(core_axis_name="c", subcore_axis_name="s")


@functools.partial(
    pl.kernel,
    out_type=(
        jax.ShapeDtypeStruct((NC, N_NODES, HW), jnp.float32),
        jax.ShapeDtypeStruct((NC, N_NODES, DEG_W), jnp.float32),
    ),
    mesh=_mesh,
    compiler_params=pltpu.CompilerParams(use_tc_tiling_on_sc=False),
    scratch_types=[
        [pltpu.VMEM((G, K), jnp.int32) for _ in range(NIB)],
        [pltpu.VMEM((G, K), jnp.int32) for _ in range(NIB)],
        [pltpu.VMEM((K, HW), jnp.float32) for _ in range(NBUF)],
        pltpu.VMEM((K, DEG_W), jnp.float32),
        pltpu.VMEM_SHARED((N_NODES, HW), jnp.float32),
        pltpu.VMEM_SHARED((N_ACC, HW), jnp.float32),
        pltpu.VMEM_SHARED((N_ACC, DEG_W), jnp.float32),
        [pltpu.SemaphoreType.DMA for _ in range(NIB)],
        [pltpu.SemaphoreType.DMA for _ in range(NIB)],
        [pltpu.SemaphoreType.DMA for _ in range(NBUF)],
        [pltpu.SemaphoreType.DMA for _ in range(NBUF)],
        [pltpu.SemaphoreType.DMA for _ in range(NBUF)],
    ],
)
def _sc_agg(x_hbm, e_hbm, zrow_hbm, zdeg_hbm, ones_hbm,
            out_hbm, deg_out_hbm,
            ib_s, ib_d, rows_v, ones_v, x_sh, acc_sh, deg_sh,
            sem_is, sem_id, sem_g, sem_s, sem_d):
    c = lax.axis_index("c")
    s = lax.axis_index("s")

    pltpu.sync_copy(ones_hbm, ones_v)

    @pl.when(s < NS - 1)
    def _():
        pltpu.sync_copy(x_hbm.at[c, pl.ds(s * WRB, WRB)],
                        x_sh.at[pl.ds(s * WRB, WRB)])
        pltpu.sync_copy(zrow_hbm, acc_sh.at[pl.ds(s * WRB, WRB)])
        pltpu.sync_copy(zdeg_hbm, deg_sh.at[pl.ds(s * WRB, WRB)])

    @pl.when(s == NS - 1)
    def _():
        pltpu.sync_copy(x_hbm.at[c, pl.ds((NS - 1) * WRB, WRB_LAST)],
                        x_sh.at[pl.ds((NS - 1) * WRB, WRB_LAST)])
        pltpu.sync_copy(zrow_hbm.at[pl.ds(0, WRB_LAST)],
                        acc_sh.at[pl.ds((NS - 1) * WRB, WRB_LAST)])
        pltpu.sync_copy(zdeg_hbm.at[pl.ds(0, WRB_LAST)],
                        deg_sh.at[pl.ds((NS - 1) * WRB, WRB_LAST)])

    plsc.subcore_barrier()

    def idx_copies(g, b):
        return (
            pltpu.make_async_copy(e_hbm.at[s * NG + g], ib_s[b], sem_is[b]),
            pltpu.make_async_copy(e_hbm.at[(NS + s) * NG + g], ib_d[b],
                                  sem_id[b]),
        )

    def idx_start(g, b):
        for cp in idx_copies(g, b):
            cp.start()

    def idx_wait(g, b):
        for cp in idx_copies(g, b):
            cp.wait()

    def gather(i, t):
        b, u = i // G, i % NBUF
        pltpu.async_copy(x_sh.at[ib_s[b].at[i % G]], rows_v[u], sem_g[u])

    def gwait(i, t):
        b, u = i // G, i % NBUF
        pltpu.make_async_copy(x_sh.at[ib_s[b].at[i % G]], rows_v[u],
                              sem_g[u]).wait()

    def deg_on(i):
        return (c == 0) if i % 2 == 0 else (c != 0)

    def scat(i, t):
        b, u = i // G, i % NBUF
        pltpu.async_copy(rows_v[u], acc_sh.at[ib_d[b].at[i % G]],
                         sem_s[u], add=True)

        @pl.when(deg_on(i))
        def _():
            pltpu.async_copy(ones_v, deg_sh.at[ib_d[b].at[i % G]],
                             sem_d[u], add=True)

    def swait(i, t):
        b, u = i // G, i % NBUF
        pltpu.make_async_copy(rows_v[u], acc_sh.at[ib_d[b].at[i % G]],
                              sem_s[u]).wait()

        @pl.when(deg_on(i))
        def _():
            pltpu.make_async_copy(ones_v, deg_sh.at[ib_d[b].at[i % G]],
                                  sem_d[u]).wait()

    for b0 in range(NIB - 1):
        idx_start(b0, b0)

    def body(t, carry):
        for b in range(NIB - 1):
            idx_wait(t * NIB + b, b)

        for i0 in range(NBUF - 1):
            gather(i0, t)

        for i in range(CPI):
            gwait(i, t)
            scat(i, t)

            if i == 0:
                @pl.when(t >= 1)
                def _():
                    swait(CPI - 1, t)

                idx_start(t * NIB + (NIB - 1), NIB - 1)
            else:
                swait(i - 1, t)

            if i == 20:
                idx_wait(t * NIB + (NIB - 1), NIB - 1)

            if i + NBUF - 1 < CPI:
                gather(i + NBUF - 1, t)

        @pl.when(t < NITER - 1)
        def _():
            for b in range(NIB - 1):
                idx_start((t + 1) * NIB + b, b)

        return carry

    lax.fori_loop(0, NITER, body, 0)

    swait(CPI - 1, NITER - 1)

    plsc.subcore_barrier()

    @pl.when(s < NS - 1)
    def _():
        pltpu.sync_copy(acc_sh.at[pl.ds(s * WRB, WRB)],
                        out_hbm.at[c, pl.ds(s * WRB, WRB)])
        pltpu.sync_copy(deg_sh.at[pl.ds(s * WRB, WRB)],
                        deg_out_hbm.at[c, pl.ds(s * WRB, WRB)])

    @pl.when(s == NS - 1)
    def _():
        pltpu.sync_copy(acc_sh.at[pl.ds((NS - 1) * WRB, WRB_LAST)],
                        out_hbm.at[c, pl.ds((NS - 1) * WRB, WRB_LAST)])
        pltpu.sync_copy(deg_sh.at[pl.ds((NS - 1) * WRB, WRB_LAST)],
                        deg_out_hbm.at[c, pl.ds((NS - 1) * WRB, WRB_LAST)])


_ROWS_BLK = 5000


def _combine_body(p_ref, d_ref, o_ref):
    deg = d_ref[0, :, 0] + d_ref[1, :, 0]
    inv = 1.0 / jnp.clip(deg, 1.0, None)[:, None]
    o_ref[...] = jnp.concatenate([p_ref[0], p_ref[1]], axis=-1) * inv


def _combine(partial, deg8):
    return pl.pallas_call(
        _combine_body,
        out_shape=jax.ShapeDtypeStruct((N_NODES, D_FEAT), jnp.float32),
        grid=(N_NODES // _ROWS_BLK,),
        in_specs=[
            pl.BlockSpec((NC, _ROWS_BLK, HW), lambda i: (0, i, 0)),
            pl.BlockSpec((NC, _ROWS_BLK, DEG_W), lambda i: (0, i, 0)),
        ],
        out_specs=pl.BlockSpec((_ROWS_BLK, D_FEAT), lambda i: (i, 0)),
    )(partial, deg8)


def kernel(x, edge_index):
    x2 = x.reshape(N_NODES, NC, HW).transpose(1, 0, 2)

    e_slab = edge_index.reshape(NC * NS * NG, G, K)

    zrow = jnp.zeros((WRB, HW), jnp.float32)
    zdeg = jnp.zeros((WRB, DEG_W), jnp.float32)
    ones = jnp.ones((K, DEG_W), jnp.float32)
    partial, deg8 = _sc_agg(x2, e_slab, zrow, zdeg, ones)
    return _combine(partial, deg8)
